# baseline (speedup 1.0000x reference)
import functools

import jax
import jax.numpy as jnp
from jax import lax
from jax.experimental import pallas as pl
from jax.experimental.pallas import tpu as pltpu
from jax.experimental.pallas import tpu_sc as plsc

N_NODES = 10000
D = 128
DH = D // 2
N_EDGES = 320000

NUM_CORES = 2
NUM_SUBCORES = 16
NUM_TILES = NUM_CORES * NUM_SUBCORES

NPAD = 10112
ROWS_PER_TILE = NPAD // NUM_SUBCORES
DUMMY_DST = N_NODES

EPAD = 327680
E_PER_TILE = EPAD // NUM_TILES
CHUNK = 128
N_CHUNKS = E_PER_TILE // CHUNK
NBUF = 3
N_STEPS = N_CHUNKS + 1
N_ITERS = N_STEPS // NBUF


def _make_seg_body(compute_deg, in_rows):
    def body(*refs):
        if compute_deg:
            (x_hbm, src_hbm, dst_hbm, z2d_hbm, z1d_hbm, ones_hbm,
             s_out, deg_out,
             src_idx, dst_idx, rv0, rv1, rv2, ones_v,
             xsp_sh, acc_sh, deg_sh, gsem, ssem, dsem) = refs
        else:
            (x_hbm, src_hbm, dst_hbm, z2d_hbm, s_out,
             src_idx, dst_idx, rv0, rv1, rv2,
             xsp_sh, acc_sh, gsem, ssem) = refs
        rv = (rv0, rv1, rv2)
        c = lax.axis_index("c")
        s = lax.axis_index("s")
        tid = c * NUM_SUBCORES + s
        rbase = s * ROWS_PER_TILE

        pltpu.sync_copy(src_hbm.at[pl.ds(tid * N_CHUNKS, N_CHUNKS)], src_idx)
        pltpu.sync_copy(dst_hbm.at[pl.ds(tid * N_CHUNKS, N_CHUNKS)], dst_idx)
        if compute_deg:
            pltpu.sync_copy(z1d_hbm, deg_sh.at[pl.ds(rbase, ROWS_PER_TILE)])
            pltpu.sync_copy(ones_hbm, ones_v)

        last_rows = in_rows - (NUM_SUBCORES - 1) * ROWS_PER_TILE

        for p in range(2):
            deg_pass = compute_deg and p == 0
            if last_rows == ROWS_PER_TILE:
                pltpu.sync_copy(
                    x_hbm.at[pl.ds(rbase, ROWS_PER_TILE), pl.ds(p * DH, DH)],
                    xsp_sh.at[pl.ds(rbase, ROWS_PER_TILE)])
            else:
                @pl.when(s < NUM_SUBCORES - 1)
                def _():
                    pltpu.sync_copy(
                        x_hbm.at[pl.ds(rbase, ROWS_PER_TILE),
                                 pl.ds(p * DH, DH)],
                        xsp_sh.at[pl.ds(rbase, ROWS_PER_TILE)])
                @pl.when(s == NUM_SUBCORES - 1)
                def _():
                    lb = (NUM_SUBCORES - 1) * ROWS_PER_TILE
                    pltpu.sync_copy(
                        x_hbm.at[pl.ds(lb, last_rows), pl.ds(p * DH, DH)],
                        xsp_sh.at[pl.ds(lb, last_rows)])
            pltpu.sync_copy(z2d_hbm, acc_sh.at[pl.ds(rbase, ROWS_PER_TILE)])
            plsc.subcore_barrier()

            def step(k, u):
                j = k * NBUF + u
                b = rv[u]
                up = (u - 1) % NBUF

                @pl.when(k > 0)
                def _():
                    pltpu.make_async_copy(b, acc_sh.at[dst_idx.at[0]],
                                          ssem.at[u]).wait()
                @pl.when(j < N_CHUNKS)
                def _():
                    pltpu.async_copy(xsp_sh.at[src_idx.at[j]], b, gsem.at[u])

                i = j - 1
                bp = rv[up]
                @pl.when(i >= 0)
                def _():
                    pltpu.make_async_copy(xsp_sh.at[src_idx.at[i]], bp,
                                          gsem.at[up]).wait()
                    pltpu.async_copy(bp, acc_sh.at[dst_idx.at[i]],
                                     ssem.at[up], add=True)
                    if deg_pass:
                        pltpu.async_copy(ones_v, deg_sh.at[dst_idx.at[i]],
                                         dsem, add=True)

            def it(k, carry):
                for u in range(NBUF):
                    step(k, u)
                return carry

            lax.fori_loop(0, N_ITERS, it, 0)
            for u in range(NBUF):
                if sum(1 for i in range(N_CHUNKS) if i % NBUF == u) > N_ITERS - 1:
                    pltpu.make_async_copy(rv[u], acc_sh.at[dst_idx.at[0]],
                                          ssem.at[u]).wait()
            if compute_deg and p == 1:
                def dwait(i, carry):
                    pltpu.make_async_copy(ones_v, deg_sh.at[dst_idx.at[0]],
                                          dsem).wait()
                    return carry
                lax.fori_loop(0, N_CHUNKS, dwait, 0)
            plsc.subcore_barrier()

            pltpu.sync_copy(acc_sh.at[pl.ds(rbase, ROWS_PER_TILE)],
                            s_out.at[c, p, pl.ds(rbase, ROWS_PER_TILE)])
            if compute_deg and p == 1:
                pltpu.sync_copy(deg_sh.at[pl.ds(rbase, ROWS_PER_TILE)],
                                deg_out.at[c, pl.ds(rbase, ROWS_PER_TILE)])
            plsc.subcore_barrier()

    return body


def _make_seg(compute_deg, in_rows):
    out_type = [jax.ShapeDtypeStruct((NUM_CORES, 2, NPAD, DH), jnp.float32)]
    if compute_deg:
        out_type.append(jax.ShapeDtypeStruct((NUM_CORES, NPAD), jnp.float32))
    scratch = [
        pltpu.VMEM((N_CHUNKS, CHUNK), jnp.int32),
        pltpu.VMEM((N_CHUNKS, CHUNK), jnp.int32),
        pltpu.VMEM((CHUNK, DH), jnp.float32),
        pltpu.VMEM((CHUNK, DH), jnp.float32),
        pltpu.VMEM((CHUNK, DH), jnp.float32),
    ]
    if compute_deg:
        scratch.append(pltpu.VMEM((CHUNK,), jnp.float32))
    scratch.append(pltpu.VMEM_SHARED((NPAD, DH), jnp.float32))
    scratch.append(pltpu.VMEM_SHARED((NPAD, DH), jnp.float32))
    if compute_deg:
        scratch.append(pltpu.VMEM_SHARED((NPAD,), jnp.float32))
    scratch += [
        pltpu.SemaphoreType.DMA((NBUF,)),
        pltpu.SemaphoreType.DMA((NBUF,)),
    ]
    if compute_deg:
        scratch.append(pltpu.SemaphoreType.DMA)
    return pl.kernel(
        _make_seg_body(compute_deg, in_rows),
        out_type=out_type,
        scratch_types=scratch,
        mesh=plsc.VectorSubcoreMesh(core_axis_name="c", subcore_axis_name="s"),
        compiler_params=pltpu.CompilerParams(use_tc_tiling_on_sc=False,
                                             needs_layout_passes=False),
    )


_seg_sum_deg = _make_seg(True, N_NODES)
_seg_sum = _make_seg(False, N_NODES)


def _dense_body(relu, s_ref, degt_ref, x_ref, wl_ref, b_ref, wr_ref, o_ref):
    deg = degt_ref[:, 0:1] + degt_ref[:, 1:2]
    inv = 1.0 / jnp.maximum(deg, 1.0)
    agg = jnp.concatenate(
        [s_ref[0, 0] + s_ref[1, 0], s_ref[0, 1] + s_ref[1, 1]],
        axis=1) * inv
    y = (jnp.dot(agg, wl_ref[...], preferred_element_type=jnp.float32)
         + b_ref[...]
         + jnp.dot(x_ref[...], wr_ref[...], preferred_element_type=jnp.float32))
    o_ref[...] = jnp.maximum(y, 0.0) if relu else y


def _dense(s, degt, x, w_l, b, w_r, relu, bm, rows):
    grid = (rows // bm,)
    out_shape = jax.ShapeDtypeStruct((rows, D), jnp.float32)
    out_specs = pl.BlockSpec((bm, D), lambda i: (i, 0))
    return pl.pallas_call(
        functools.partial(_dense_body, relu),
        grid=grid,
        in_specs=[
            pl.BlockSpec((NUM_CORES, 2, bm, DH), lambda i: (0, 0, i, 0)),
            pl.BlockSpec((bm, NUM_CORES), lambda i: (i, 0)),
            pl.BlockSpec((bm, D), lambda i: (i, 0)),
            pl.BlockSpec((D, D), lambda i: (0, 0)),
            pl.BlockSpec((1, D), lambda i: (0, 0)),
            pl.BlockSpec((D, D), lambda i: (0, 0)),
        ],
        out_specs=out_specs,
        out_shape=out_shape,
        compiler_params=pltpu.CompilerParams(
            dimension_semantics=("arbitrary",)),
    )(s, degt, x, w_l, b.reshape(1, D), w_r)


def kernel(x, edge_index, W1_l, b1, W1_r, W2_l, b2, W2_r):
    src = edge_index[0].astype(jnp.int32)
    dst = edge_index[1].astype(jnp.int32)
    src = jnp.concatenate([src, jnp.zeros((EPAD - N_EDGES,), jnp.int32)])
    dst = jnp.concatenate([dst, jnp.full((EPAD - N_EDGES,), DUMMY_DST, jnp.int32)])
    src2d = src.reshape(NUM_TILES * N_CHUNKS, CHUNK)
    dst2d = dst.reshape(NUM_TILES * N_CHUNKS, CHUNK)

    z2d = jnp.zeros((ROWS_PER_TILE, DH), jnp.float32)
    z1d = jnp.zeros((ROWS_PER_TILE,), jnp.float32)
    ones = jnp.ones((CHUNK,), jnp.float32)

    s1, degp = _seg_sum_deg(x, src2d, dst2d, z2d, z1d, ones)
    degt = degp.T
    h = _dense(s1, degt, x, W1_l, b1, W1_r, relu=True, bm=2000, rows=N_NODES)
    (s2,) = _seg_sum(h, src2d, dst2d, z2d)
    return _dense(s2, degt, h, W2_l, b2, W2_r, relu=False, bm=2000,
                  rows=N_NODES)
